# baseline (speedup 1.0000x reference)
import jax
import jax.numpy as jnp
from jax import lax
from jax.experimental import pallas as pl
from jax.experimental.pallas import tpu as pltpu
from jax.experimental.pallas import tpu_sc as plsc

B = 4096
T_IN = 1024
N_IN = 10
N_ST = 6
N_OUT = 64
K_IN = 20
NPAT = 1024
NZ = 64
NW = 32
BPW = B // NW
PPW = NPAT // NW

_f32 = jnp.float32
_i32 = jnp.int32


_GRID_B = 16
_BBLK = B // _GRID_B
_WPN = 1 << 16
_KPS = 16 // _GRID_B


def _dot(a, b, dims):
    return lax.dot_general(a, b, (dims, ((), ())),
                           preferred_element_type=_f32,
                           precision=lax.Precision.HIGHEST)


def _addr_body(memin_ref, bits_ref, ci_ref, cs_ref, co_ref,
               packed_ref, inidx_ref, st_ref, aout_ref, sm_ref, wt_ref):
    i = pl.program_id(0)

    @pl.when(i == 0)
    def _():
        ci = ci_ref[...]
        n3 = lax.broadcasted_iota(_i32, (16, K_IN, T_IN), 0)
        k3 = lax.broadcasted_iota(_i32, (16, K_IN, T_IN), 1)
        t3 = lax.broadcasted_iota(_i32, (16, K_IN, T_IN), 2)
        oh = (ci[:, :, None] == t3) & (n3 < N_IN)
        for sgrp in range(3):
            ohw = oh & (k3 >= sgrp * 7) & (k3 < sgrp * 7 + 7)
            wt_ref[sgrp, :, :] = jnp.sum(
                jnp.where(ohw, 1 << k3, 0), axis=1).astype(jnp.bfloat16)

        tt = lax.broadcasted_iota(_i32, (16, NPAT), 0)
        pp = lax.broadcasted_iota(_i32, (16, NPAT), 1)
        pat = ((pp >> tt) & 1).astype(_f32)

        cs = cs_ref[...]
        j3s = lax.broadcasted_iota(_i32, (8, 16, 16), 0)
        k3s = lax.broadcasted_iota(_i32, (8, 16, 16), 1)
        t3s = lax.broadcasted_iota(_i32, (8, 16, 16), 2)
        ohs = (cs[:, :, None] == t3s) & (t3s < N_IN) & (j3s < N_ST)
        ws = jnp.sum(jnp.where(ohs, 1 << k3s, 0), axis=1).astype(_f32)
        srow = lax.broadcasted_iota(_i32, (8, NPAT), 0)
        st_ref[...] = (_dot(ws, pat, ((1,), (0,))) + 0.5).astype(_i32) + (
            jnp.where(srow < N_ST, srow << 16, 0))

        co = co_ref[...]
        k3o = lax.broadcasted_iota(_i32, (N_OUT, 16, 16), 1)
        t3o = lax.broadcasted_iota(_i32, (N_OUT, 16, 16), 2)
        c3o = jnp.broadcast_to(co[:, :, None], (N_OUT, 16, 16))
        oh_lo = (c3o == t3o) & (c3o < N_IN)
        wa = jnp.sum(jnp.where(oh_lo, 1 << k3o, 0), axis=1).astype(_f32)
        arow = lax.broadcasted_iota(_i32, (N_OUT, NPAT), 0)
        aout_ref[...] = (_dot(wa, pat, ((1,), (0,))) + 0.5).astype(_i32) + (
            arow << 16)

        k3m = lax.broadcasted_iota(_i32, (N_OUT, 16, 8), 1)
        s3m = lax.broadcasted_iota(_i32, (N_OUT, 16, 8), 2)
        c3m = jnp.broadcast_to(co[:, :, None], (N_OUT, 16, 8))
        oh_hi = ((c3m - N_IN) == s3m) & (c3m >= N_IN)
        m = jnp.sum(jnp.where(oh_hi, 1 << k3m, 0), axis=1).astype(_f32)

        ss = lax.broadcasted_iota(_i32, (8, NZ), 0)
        zz = lax.broadcasted_iota(_i32, (8, NZ), 1)
        zb = ((zz >> ss) & 1).astype(_f32)
        sm_ref[...] = (_dot(m, zb, ((1,), (0,))) + 0.5).astype(_i32)

    bits = bits_ref[...].astype(jnp.bfloat16)
    prod = jnp.zeros((16, _BBLK), _f32)
    for sgrp in range(3):
        prod = prod + lax.dot_general(
            wt_ref[sgrp, :, :], bits, (((1,), (1,)), ((), ())),
            preferred_element_type=_f32)
    row = lax.broadcasted_iota(_i32, (16, _BBLK), 0)
    offs = jnp.where(row < N_IN, row << 20, 0)
    inidx_ref[...] = (prod + 0.5).astype(_i32) + offs

    for n in range(N_IN):
        acc = jnp.zeros((_WPN,), _i32)
        for kk in range(_KPS):
            v = memin_ref[n, pl.ds(kk * _WPN, _WPN)]
            acc = acc | jnp.where(v > 0.5,
                                  jnp.int32(1) << (i * _KPS + kk), 0)

        @pl.when(i == 0)
        def _():
            packed_ref[pl.ds(n * _WPN, _WPN)] = acc

        @pl.when(i > 0)
        def _():
            packed_ref[pl.ds(n * _WPN, _WPN)] = (
                packed_ref[pl.ds(n * _WPN, _WPN)] | acc)


def _addr_call(mem_in, input_bits, ci, cs, co):
    return pl.pallas_call(
        _addr_body,
        grid=(_GRID_B,),
        in_specs=[
            pl.BlockSpec((N_IN, _KPS * _WPN), lambda i: (0, i)),
            pl.BlockSpec((_BBLK, T_IN), lambda i: (i, 0)),
            pl.BlockSpec((16, K_IN), lambda i: (0, 0)),
            pl.BlockSpec((8, 16), lambda i: (0, 0)),
            pl.BlockSpec((N_OUT, 16), lambda i: (0, 0)),
        ],
        out_specs=(
            pl.BlockSpec((N_IN * _WPN,), lambda i: (0,)),
            pl.BlockSpec((16, _BBLK), lambda i: (0, i)),
            pl.BlockSpec((8, NPAT), lambda i: (0, 0)),
            pl.BlockSpec((N_OUT, NPAT), lambda i: (0, 0)),
            pl.BlockSpec((N_OUT, NZ), lambda i: (0, 0)),
        ),
        out_shape=(
            jax.ShapeDtypeStruct((N_IN * _WPN,), _i32),
            jax.ShapeDtypeStruct((16, B), _i32),
            jax.ShapeDtypeStruct((8, NPAT), _i32),
            jax.ShapeDtypeStruct((N_OUT, NPAT), _i32),
            jax.ShapeDtypeStruct((N_OUT, NZ), _i32),
        ),
        scratch_shapes=[pltpu.VMEM((3, 16, T_IN), jnp.bfloat16)],
        compiler_params=pltpu.CompilerParams(
            dimension_semantics=("arbitrary",)),
    )(mem_in, input_bits, ci, cs, co)


_RCHUNK = 4096


def _sc_relayout_body(memout_hbm, oflat_hbm, slab_v):
    c = lax.axis_index("c")
    s = lax.axis_index("s")
    w = s * 2 + c
    tr = w % 8
    cq = w // 8
    for chunk in range(4):
        c0 = cq * 16384 + chunk * _RCHUNK
        pltpu.sync_copy(memout_hbm.at[pl.ds(tr * 8, 8), pl.ds(c0, _RCHUNK)],
                        slab_v)
        for r in range(8):
            row = tr * 8 + r
            off = pl.multiple_of((row << 16) + c0, _RCHUNK)
            pltpu.sync_copy(slab_v.at[r], oflat_hbm.at[pl.ds(off, _RCHUNK)])


def _sc_relayout_call(mem_out):
    mesh = plsc.VectorSubcoreMesh(core_axis_name="c", subcore_axis_name="s")
    f = pl.kernel(
        _sc_relayout_body,
        out_type=jax.ShapeDtypeStruct((N_OUT << 16,), _f32),
        mesh=mesh,
        compiler_params=pltpu.CompilerParams(use_tc_tiling_on_sc=True,
                                             needs_layout_passes=False),
        scratch_types=[
            pltpu.VMEM((8, _RCHUNK), _f32),
        ],
    )
    return f(mem_out)


def _sc_main_body(inidx_hbm, staddr_hbm, aout_hbm, sm_hbm,
                  packed_hbm, memst_hbm, memout_hbm,
                  p_hbm, table_hbm,
                  idx_in_v, word_v, g_v, p_v, st_idx_v, st_vals_v, z_v,
                  a_chunk_v, sm_v, oidx_v, tvals_v, sem):
    c = lax.axis_index("c")
    s = lax.axis_index("s")
    w = s * 2 + c
    b0 = w * BPW
    t0 = w * PPW

    pltpu.sync_copy(inidx_hbm.at[pl.ds(0, N_IN), pl.ds(b0, BPW)], idx_in_v)
    pltpu.sync_copy(staddr_hbm.at[pl.ds(0, N_ST), pl.ds(t0, PPW)], st_idx_v)
    pltpu.sync_copy(aout_hbm.at[pl.ds(0, N_OUT), pl.ds(t0, PPW)], a_chunk_v)
    pltpu.sync_copy(sm_hbm, sm_v)

    for blk in range(BPW // 16):
        for n in range(N_IN):
            f = idx_in_v[n, pl.ds(blk * 16, 16)]
            g_v[n, pl.ds(blk * 16, 16)] = (
                ((f >> 20) << 16) | (f & jnp.int32(0xFFFF)))

    cps = [pltpu.async_copy(packed_hbm.at[g_v.at[n]], word_v.at[n], sem)
           for n in range(N_IN)]
    cps += [pltpu.async_copy(memst_hbm.at[st_idx_v.at[j]],
                             st_vals_v.at[j], sem)
            for j in range(N_ST)]
    for cp in cps:
        cp.wait()

    for blk in range(BPW // 16):
        acc = jnp.zeros((16,), _i32)
        for n in range(N_IN):
            f = idx_in_v[n, pl.ds(blk * 16, 16)]
            word = word_v[n, pl.ds(blk * 16, 16)]
            bit = (word >> ((f >> 16) & 15)) & 1
            acc = acc | jnp.where(bit > 0, jnp.int32(1 << n), jnp.int32(0))
        p_v[pl.ds(blk * 16, 16)] = acc
    pltpu.sync_copy(p_v, p_hbm.at[pl.ds(b0, BPW)])

    for blk in range(PPW // 16):
        z = jnp.zeros((16,), _i32)
        for j in range(N_ST):
            v = st_vals_v[j, pl.ds(blk * 16, 16)]
            z = z | jnp.where(v > 0.5, jnp.int32(1 << j), jnp.int32(0))
        z_v[pl.ds(blk * 16, 16)] = z

    lane = lax.iota(_i32, 16)
    for blk in range(PPW // 16):
        z16 = z_v[pl.ds(blk * 16, 16)]
        pl16 = lane + blk * 16
        for j3 in range(N_OUT):
            smv = plsc.load_gather(sm_v, [jnp.full((16,), j3, _i32), z16])
            addr16 = a_chunk_v[j3, pl.ds(blk * 16, 16)] + smv
            flat16 = pl16 * N_OUT + j3
            plsc.store_scatter(oidx_v, [flat16 >> 7, flat16 & 127], addr16)
    cps = [pltpu.async_copy(memout_hbm.at[oidx_v.at[r]],
                            tvals_v.at[pl.ds(r * 128, 128)], sem)
           for r in range(PPW * N_OUT // 128)]
    for cp in cps:
        cp.wait()
    pltpu.sync_copy(tvals_v, table_hbm.at[pl.ds(t0 * N_OUT, PPW * N_OUT)])


def _sc_main_call(inidx, staddr, aout, sm, packed, msflat, moflat):
    mesh = plsc.VectorSubcoreMesh(core_axis_name="c", subcore_axis_name="s")
    f = pl.kernel(
        _sc_main_body,
        out_type=(
            jax.ShapeDtypeStruct((B,), _i32),
            jax.ShapeDtypeStruct((NPAT * N_OUT,), _f32),
        ),
        mesh=mesh,
        compiler_params=pltpu.CompilerParams(use_tc_tiling_on_sc=False,
                                             needs_layout_passes=False),
        scratch_types=[
            pltpu.VMEM((N_IN, BPW), _i32),
            pltpu.VMEM((N_IN, BPW), _i32),
            pltpu.VMEM((N_IN, BPW), _i32),
            pltpu.VMEM((BPW,), _i32),
            pltpu.VMEM((N_ST, PPW), _i32),
            pltpu.VMEM((N_ST, PPW), _f32),
            pltpu.VMEM((PPW,), _i32),
            pltpu.VMEM((N_OUT, PPW), _i32),
            pltpu.VMEM((N_OUT, NZ), _i32),
            pltpu.VMEM((16, 128), _i32),
            pltpu.VMEM((PPW * N_OUT,), _f32),
            pltpu.SemaphoreType.DMA,
        ],
    )
    return f(inidx, staddr, aout, sm, packed, msflat, moflat)


def _sc_join_body(table_hbm, p_hbm, out_hbm, idx_v, rows_v, sem):
    c = lax.axis_index("c")
    s = lax.axis_index("s")
    w = s * 2 + c
    b0 = w * BPW
    pltpu.sync_copy(p_hbm.at[pl.ds(b0, BPW)], idx_v)
    pltpu.async_copy(table_hbm.at[idx_v], rows_v, sem).wait()
    pltpu.sync_copy(rows_v, out_hbm.at[pl.ds(b0, BPW)])


def _sc_join_call(table2d, p):
    mesh = plsc.VectorSubcoreMesh(core_axis_name="c", subcore_axis_name="s")
    f = pl.kernel(
        _sc_join_body,
        out_type=jax.ShapeDtypeStruct((B, N_OUT), _f32),
        mesh=mesh,
        compiler_params=pltpu.CompilerParams(use_tc_tiling_on_sc=False,
                                             needs_layout_passes=False),
        scratch_types=[
            pltpu.VMEM((BPW,), _i32),
            pltpu.VMEM((BPW, N_OUT), _f32),
            pltpu.SemaphoreType.DMA,
        ],
    )
    return f(table2d, p)


def kernel(input_bits, conn_in, mem_in, conn_state, mem_state, conn_out,
           mem_out):
    ci = jnp.pad(conn_in, ((0, 16 - N_IN), (0, 0)))
    cs = jnp.pad(conn_state, ((0, 8 - N_ST), (0, 0)))

    packed, inidx, staddr, aout, sm = _addr_call(
        mem_in, input_bits, ci, cs, conn_out)
    moflat = _sc_relayout_call(mem_out)
    msflat = mem_state.reshape(-1)

    p, table = _sc_main_call(inidx, staddr, aout, sm, packed, msflat, moflat)

    return _sc_join_call(table.reshape(NPAT, N_OUT), p)


# speedup vs baseline: 1.0192x; 1.0192x over previous
import jax
import jax.numpy as jnp
from jax import lax
from jax.experimental import pallas as pl
from jax.experimental.pallas import tpu as pltpu
from jax.experimental.pallas import tpu_sc as plsc

B = 4096
T_IN = 1024
N_IN = 10
N_ST = 6
N_OUT = 64
K_IN = 20
NPAT = 1024
NZ = 64
NW = 32
BPW = B // NW
PPW = NPAT // NW

_f32 = jnp.float32
_i32 = jnp.int32


_GRID_B = 8
_BBLK = B // _GRID_B
_WPN = 1 << 16
_KPS = 16 // _GRID_B


def _dot(a, b, dims):
    return lax.dot_general(a, b, (dims, ((), ())),
                           preferred_element_type=_f32,
                           precision=lax.Precision.HIGHEST)


def _addr_body(memin_ref, bits_ref, ci_ref, cs_ref, co_ref,
               packed_ref, inidx_ref, st_ref, aout_ref, sm_ref, wt_ref):
    i = pl.program_id(0)

    @pl.when(i == 0)
    def _():
        ci = ci_ref[...]
        n3 = lax.broadcasted_iota(_i32, (16, K_IN, T_IN), 0)
        k3 = lax.broadcasted_iota(_i32, (16, K_IN, T_IN), 1)
        t3 = lax.broadcasted_iota(_i32, (16, K_IN, T_IN), 2)
        oh = (ci[:, :, None] == t3) & (n3 < N_IN)
        for sgrp in range(3):
            ohw = oh & (k3 >= sgrp * 7) & (k3 < sgrp * 7 + 7)
            wt_ref[sgrp, :, :] = jnp.sum(
                jnp.where(ohw, 1 << k3, 0), axis=1).astype(jnp.bfloat16)

        tt = lax.broadcasted_iota(_i32, (16, NPAT), 0)
        pp = lax.broadcasted_iota(_i32, (16, NPAT), 1)
        pat = ((pp >> tt) & 1).astype(_f32)

        cs = cs_ref[...]
        j3s = lax.broadcasted_iota(_i32, (8, 16, 16), 0)
        k3s = lax.broadcasted_iota(_i32, (8, 16, 16), 1)
        t3s = lax.broadcasted_iota(_i32, (8, 16, 16), 2)
        ohs = (cs[:, :, None] == t3s) & (t3s < N_IN) & (j3s < N_ST)
        ws = jnp.sum(jnp.where(ohs, 1 << k3s, 0), axis=1).astype(_f32)
        srow = lax.broadcasted_iota(_i32, (8, NPAT), 0)
        st_ref[...] = (_dot(ws, pat, ((1,), (0,))) + 0.5).astype(_i32) + (
            jnp.where(srow < N_ST, srow << 16, 0))

        co = co_ref[...]
        k3o = lax.broadcasted_iota(_i32, (N_OUT, 16, 16), 1)
        t3o = lax.broadcasted_iota(_i32, (N_OUT, 16, 16), 2)
        c3o = jnp.broadcast_to(co[:, :, None], (N_OUT, 16, 16))
        oh_lo = (c3o == t3o) & (c3o < N_IN)
        wa = jnp.sum(jnp.where(oh_lo, 1 << k3o, 0), axis=1).astype(_f32)
        arow = lax.broadcasted_iota(_i32, (N_OUT, NPAT), 0)
        aout_ref[...] = (_dot(wa, pat, ((1,), (0,))) + 0.5).astype(_i32) + (
            arow << 16)

        k3m = lax.broadcasted_iota(_i32, (N_OUT, 16, 8), 1)
        s3m = lax.broadcasted_iota(_i32, (N_OUT, 16, 8), 2)
        c3m = jnp.broadcast_to(co[:, :, None], (N_OUT, 16, 8))
        oh_hi = ((c3m - N_IN) == s3m) & (c3m >= N_IN)
        m = jnp.sum(jnp.where(oh_hi, 1 << k3m, 0), axis=1).astype(_f32)

        ss = lax.broadcasted_iota(_i32, (8, NZ), 0)
        zz = lax.broadcasted_iota(_i32, (8, NZ), 1)
        zb = ((zz >> ss) & 1).astype(_f32)
        sm_ref[...] = (_dot(m, zb, ((1,), (0,))) + 0.5).astype(_i32)

    bits = bits_ref[...].astype(jnp.bfloat16)
    prod = jnp.zeros((16, _BBLK), _f32)
    for sgrp in range(3):
        prod = prod + lax.dot_general(
            wt_ref[sgrp, :, :], bits, (((1,), (1,)), ((), ())),
            preferred_element_type=_f32)
    row = lax.broadcasted_iota(_i32, (16, _BBLK), 0)
    offs = jnp.where(row < N_IN, row << 20, 0)
    inidx_ref[...] = (prod + 0.5).astype(_i32) + offs

    for n in range(N_IN):
        acc = jnp.zeros((_WPN,), _i32)
        for kk in range(_KPS):
            v = memin_ref[n, pl.ds(kk * _WPN, _WPN)]
            acc = acc | jnp.where(v > 0.5,
                                  jnp.int32(1) << (i * _KPS + kk), 0)

        @pl.when(i == 0)
        def _():
            packed_ref[pl.ds(n * _WPN, _WPN)] = acc

        @pl.when(i > 0)
        def _():
            packed_ref[pl.ds(n * _WPN, _WPN)] = (
                packed_ref[pl.ds(n * _WPN, _WPN)] | acc)


def _addr_call(mem_in, input_bits, ci, cs, co):
    return pl.pallas_call(
        _addr_body,
        grid=(_GRID_B,),
        in_specs=[
            pl.BlockSpec((N_IN, _KPS * _WPN), lambda i: (0, i)),
            pl.BlockSpec((_BBLK, T_IN), lambda i: (i, 0)),
            pl.BlockSpec((16, K_IN), lambda i: (0, 0)),
            pl.BlockSpec((8, 16), lambda i: (0, 0)),
            pl.BlockSpec((N_OUT, 16), lambda i: (0, 0)),
        ],
        out_specs=(
            pl.BlockSpec((N_IN * _WPN,), lambda i: (0,)),
            pl.BlockSpec((16, _BBLK), lambda i: (0, i)),
            pl.BlockSpec((8, NPAT), lambda i: (0, 0)),
            pl.BlockSpec((N_OUT, NPAT), lambda i: (0, 0)),
            pl.BlockSpec((N_OUT, NZ), lambda i: (0, 0)),
        ),
        out_shape=(
            jax.ShapeDtypeStruct((N_IN * _WPN,), _i32),
            jax.ShapeDtypeStruct((16, B), _i32),
            jax.ShapeDtypeStruct((8, NPAT), _i32),
            jax.ShapeDtypeStruct((N_OUT, NPAT), _i32),
            jax.ShapeDtypeStruct((N_OUT, NZ), _i32),
        ),
        scratch_shapes=[pltpu.VMEM((3, 16, T_IN), jnp.bfloat16)],
        compiler_params=pltpu.CompilerParams(
            dimension_semantics=("arbitrary",)),
    )(mem_in, input_bits, ci, cs, co)


_RCHUNK = 4096


def _sc_relayout_body(memout_hbm, oflat_hbm, slab_v):
    c = lax.axis_index("c")
    s = lax.axis_index("s")
    w = s * 2 + c
    tr = w % 8
    cq = w // 8
    for chunk in range(4):
        c0 = cq * 16384 + chunk * _RCHUNK
        pltpu.sync_copy(memout_hbm.at[pl.ds(tr * 8, 8), pl.ds(c0, _RCHUNK)],
                        slab_v)
        for r in range(8):
            row = tr * 8 + r
            off = pl.multiple_of((row << 16) + c0, _RCHUNK)
            pltpu.sync_copy(slab_v.at[r], oflat_hbm.at[pl.ds(off, _RCHUNK)])


def _sc_relayout_call(mem_out):
    mesh = plsc.VectorSubcoreMesh(core_axis_name="c", subcore_axis_name="s")
    f = pl.kernel(
        _sc_relayout_body,
        out_type=jax.ShapeDtypeStruct((N_OUT << 16,), _f32),
        mesh=mesh,
        compiler_params=pltpu.CompilerParams(use_tc_tiling_on_sc=True,
                                             needs_layout_passes=False),
        scratch_types=[
            pltpu.VMEM((8, _RCHUNK), _f32),
        ],
    )
    return f(mem_out)


def _sc_main_body(inidx_hbm, staddr_hbm, aout_hbm, sm_hbm,
                  packed_hbm, memst_hbm, memout_hbm,
                  p_hbm, table_hbm,
                  idx_in_v, word_v, g_v, p_v, st_idx_v, st_vals_v, z_v,
                  a_chunk_v, sm_v, oidx_v, tvals_v, sem):
    c = lax.axis_index("c")
    s = lax.axis_index("s")
    w = s * 2 + c
    b0 = w * BPW
    t0 = w * PPW

    pltpu.sync_copy(inidx_hbm.at[pl.ds(0, N_IN), pl.ds(b0, BPW)], idx_in_v)
    pltpu.sync_copy(staddr_hbm.at[pl.ds(0, N_ST), pl.ds(t0, PPW)], st_idx_v)
    pltpu.sync_copy(aout_hbm.at[pl.ds(0, N_OUT), pl.ds(t0, PPW)], a_chunk_v)
    pltpu.sync_copy(sm_hbm, sm_v)

    for blk in range(BPW // 16):
        for n in range(N_IN):
            f = idx_in_v[n, pl.ds(blk * 16, 16)]
            g_v[n, pl.ds(blk * 16, 16)] = (
                ((f >> 20) << 16) | (f & jnp.int32(0xFFFF)))

    cps = [pltpu.async_copy(packed_hbm.at[g_v.at[n]], word_v.at[n], sem)
           for n in range(N_IN)]
    cps += [pltpu.async_copy(memst_hbm.at[st_idx_v.at[j]],
                             st_vals_v.at[j], sem)
            for j in range(N_ST)]
    for cp in cps:
        cp.wait()

    for blk in range(BPW // 16):
        acc = jnp.zeros((16,), _i32)
        for n in range(N_IN):
            f = idx_in_v[n, pl.ds(blk * 16, 16)]
            word = word_v[n, pl.ds(blk * 16, 16)]
            bit = (word >> ((f >> 16) & 15)) & 1
            acc = acc | jnp.where(bit > 0, jnp.int32(1 << n), jnp.int32(0))
        p_v[pl.ds(blk * 16, 16)] = acc
    pltpu.sync_copy(p_v, p_hbm.at[pl.ds(b0, BPW)])

    for blk in range(PPW // 16):
        z = jnp.zeros((16,), _i32)
        for j in range(N_ST):
            v = st_vals_v[j, pl.ds(blk * 16, 16)]
            z = z | jnp.where(v > 0.5, jnp.int32(1 << j), jnp.int32(0))
        z_v[pl.ds(blk * 16, 16)] = z

    lane = lax.iota(_i32, 16)
    for blk in range(PPW // 16):
        z16 = z_v[pl.ds(blk * 16, 16)]
        pl16 = lane + blk * 16
        for j3 in range(N_OUT):
            smv = plsc.load_gather(sm_v, [jnp.full((16,), j3, _i32), z16])
            addr16 = a_chunk_v[j3, pl.ds(blk * 16, 16)] + smv
            flat16 = pl16 * N_OUT + j3
            plsc.store_scatter(oidx_v, [flat16 >> 7, flat16 & 127], addr16)
    cps = [pltpu.async_copy(memout_hbm.at[oidx_v.at[r]],
                            tvals_v.at[pl.ds(r * 128, 128)], sem)
           for r in range(PPW * N_OUT // 128)]
    for cp in cps:
        cp.wait()
    pltpu.sync_copy(tvals_v, table_hbm.at[pl.ds(t0 * N_OUT, PPW * N_OUT)])


def _sc_main_call(inidx, staddr, aout, sm, packed, msflat, moflat):
    mesh = plsc.VectorSubcoreMesh(core_axis_name="c", subcore_axis_name="s")
    f = pl.kernel(
        _sc_main_body,
        out_type=(
            jax.ShapeDtypeStruct((B,), _i32),
            jax.ShapeDtypeStruct((NPAT * N_OUT,), _f32),
        ),
        mesh=mesh,
        compiler_params=pltpu.CompilerParams(use_tc_tiling_on_sc=False,
                                             needs_layout_passes=False),
        scratch_types=[
            pltpu.VMEM((N_IN, BPW), _i32),
            pltpu.VMEM((N_IN, BPW), _i32),
            pltpu.VMEM((N_IN, BPW), _i32),
            pltpu.VMEM((BPW,), _i32),
            pltpu.VMEM((N_ST, PPW), _i32),
            pltpu.VMEM((N_ST, PPW), _f32),
            pltpu.VMEM((PPW,), _i32),
            pltpu.VMEM((N_OUT, PPW), _i32),
            pltpu.VMEM((N_OUT, NZ), _i32),
            pltpu.VMEM((16, 128), _i32),
            pltpu.VMEM((PPW * N_OUT,), _f32),
            pltpu.SemaphoreType.DMA,
        ],
    )
    return f(inidx, staddr, aout, sm, packed, msflat, moflat)


def _sc_join_body(table_hbm, p_hbm, out_hbm, idx_v, rows_v, sem):
    c = lax.axis_index("c")
    s = lax.axis_index("s")
    w = s * 2 + c
    b0 = w * BPW
    pltpu.sync_copy(p_hbm.at[pl.ds(b0, BPW)], idx_v)
    pltpu.async_copy(table_hbm.at[idx_v], rows_v, sem).wait()
    pltpu.sync_copy(rows_v, out_hbm.at[pl.ds(b0, BPW)])


def _sc_join_call(table2d, p):
    mesh = plsc.VectorSubcoreMesh(core_axis_name="c", subcore_axis_name="s")
    f = pl.kernel(
        _sc_join_body,
        out_type=jax.ShapeDtypeStruct((B, N_OUT), _f32),
        mesh=mesh,
        compiler_params=pltpu.CompilerParams(use_tc_tiling_on_sc=False,
                                             needs_layout_passes=False),
        scratch_types=[
            pltpu.VMEM((BPW,), _i32),
            pltpu.VMEM((BPW, N_OUT), _f32),
            pltpu.SemaphoreType.DMA,
        ],
    )
    return f(table2d, p)


def kernel(input_bits, conn_in, mem_in, conn_state, mem_state, conn_out,
           mem_out):
    ci = jnp.pad(conn_in, ((0, 16 - N_IN), (0, 0)))
    cs = jnp.pad(conn_state, ((0, 8 - N_ST), (0, 0)))

    packed, inidx, staddr, aout, sm = _addr_call(
        mem_in, input_bits, ci, cs, conn_out)
    moflat = _sc_relayout_call(mem_out)
    msflat = mem_state.reshape(-1)

    p, table = _sc_main_call(inidx, staddr, aout, sm, packed, msflat, moflat)

    return _sc_join_call(table.reshape(NPAT, N_OUT), p)
